# Initial kernel scaffold; baseline (speedup 1.0000x reference)
import functools

import jax
import jax.numpy as jnp
from jax import lax
from jax.experimental import pallas as pl
from jax.experimental.pallas import tpu as pltpu
from jax.experimental.pallas import tpu_sc as plsc

HPAD = 32
NC, NS = 2, 16
NW = NC * NS
CHUNK = 128
NBUF = 4


def _scatter_partials(npad, cpt):
    rows_pt = npad // NS

    mesh = plsc.VectorSubcoreMesh(core_axis_name="c", subcore_axis_name="s")
    scratch = [
        pltpu.VMEM_SHARED((npad, HPAD), jnp.float32),
        pltpu.VMEM((cpt, CHUNK), jnp.int32),
        pltpu.VMEM((cpt, CHUNK), jnp.int32),
        pltpu.VMEM((NBUF, CHUNK, HPAD), jnp.float32),
    ] + [pltpu.SemaphoreType.DMA] * (2 * NBUF)

    @functools.partial(
        pl.kernel,
        out_type=jax.ShapeDtypeStruct((NC, npad, HPAD), jnp.float32),
        mesh=mesh,
        scratch_types=scratch,
    )
    def kern(z_hbm, src_hbm, dst_hbm, out_hbm, acc, src_v, dst_v, rows, *sems):
        gsem = sems[:NBUF]
        ssem = sems[NBUF:]
        c = lax.axis_index("c")
        s = lax.axis_index("s")
        wid = s * NC + c
        r0 = s * rows_pt

        pltpu.sync_copy(z_hbm.at[pl.ds(c * npad + r0, rows_pt)],
                        acc.at[pl.ds(r0, rows_pt)])
        pltpu.sync_copy(src_hbm.at[wid], src_v)
        pltpu.sync_copy(dst_hbm.at[wid], dst_v)

        for b in range(NBUF):
            pltpu.async_copy(z_hbm.at[src_v.at[b]], rows.at[b], gsem[b])

        plsc.subcore_barrier()

        def step(chunk, b, prefetch):
            pltpu.make_async_copy(
                z_hbm.at[src_v.at[chunk]], rows.at[b], gsem[b]).wait()
            pltpu.async_copy(
                rows.at[b], acc.at[dst_v.at[chunk]], ssem[b], add=True)
            pltpu.make_async_copy(
                rows.at[b], acc.at[dst_v.at[chunk]], ssem[b], add=True).wait()
            if prefetch:
                pltpu.async_copy(
                    z_hbm.at[src_v.at[chunk + NBUF]], rows.at[b], gsem[b])

        def body(jo, carry):
            for b in range(NBUF):
                step(jo * NBUF + b, b, True)
            return carry

        lax.fori_loop(0, cpt // NBUF - 1, body, 0)
        for b in range(NBUF):
            step(cpt - NBUF + b, b, False)

        plsc.subcore_barrier()
        pltpu.sync_copy(acc.at[pl.ds(r0, rows_pt)],
                        out_hbm.at[c, pl.ds(r0, rows_pt)])

    return kern


def _tc_project(x, w1, npad):
    def body(x_ref, w_ref, o_ref):
        z = jnp.dot(x_ref[...], w_ref[...], preferred_element_type=jnp.float32)
        o_ref[pl.ds(0, npad), :] = z
        o_ref[pl.ds(npad, npad), :] = jnp.zeros_like(z)

    return pl.pallas_call(
        body,
        out_shape=jax.ShapeDtypeStruct((2 * npad, HPAD), jnp.float32),
    )(x, w1)


def _tc_combine(p, b1, w2, b2, w1n, npad):
    def body(p_ref, b1_ref, w2_ref, b2_ref, w1n_ref, o_ref):
        u = jnp.maximum(p_ref[0] + p_ref[1] + b1_ref[...], 0.0)
        v = jnp.dot(u, w2_ref[...], preferred_element_type=jnp.float32) + b2_ref[...]
        h = jnp.maximum(v, 0.0)
        z = jnp.dot(h, w1n_ref[...], preferred_element_type=jnp.float32)
        o_ref[pl.ds(0, npad), :] = z
        o_ref[pl.ds(npad, npad), :] = jnp.zeros_like(z)

    return pl.pallas_call(
        body,
        out_shape=jax.ShapeDtypeStruct((2 * npad, HPAD), jnp.float32),
    )(p, b1, w2, b2, w1n)


def _tc_final(p, b1, w2, b2, npad, h_real):
    def body(p_ref, b1_ref, w2_ref, b2_ref, o_ref):
        u = jnp.maximum(p_ref[0] + p_ref[1] + b1_ref[...], 0.0)
        v = jnp.dot(u, w2_ref[...], preferred_element_type=jnp.float32) + b2_ref[...]
        h = jnp.maximum(v, 0.0)
        col = lax.broadcasted_iota(jnp.int32, h.shape, 1)
        hm = jnp.where(col < h_real, h, -jnp.inf)
        m = jnp.max(hm, axis=1, keepdims=True)
        lse = m + jnp.log(jnp.sum(jnp.exp(hm - m), axis=1, keepdims=True))
        o_ref[...] = h - lse

    return pl.pallas_call(
        body,
        out_shape=jax.ShapeDtypeStruct((npad, HPAD), jnp.float32),
    )(p, b1, w2, b2)


def _pad_w(w):
    return jnp.pad(w, ((0, 0), (0, HPAD - w.shape[1])))


def _pad_w2(w):
    return jnp.pad(w, ((0, HPAD - w.shape[0]), (0, HPAD - w.shape[1])))


def _pad_b(b):
    return jnp.pad(b, (0, HPAD - b.shape[0])).reshape(1, HPAD)


def kernel(x, edge_index, W1a, b1a, W2a, b2a, W1b, b1b, W2b, b2b,
           W1c, b1c, W2c, b2c):
    n, din = x.shape
    h_real = W1a.shape[1]
    e = edge_index.shape[1]

    npad = (n // (NS * NC) + 1) * (NS * NC)
    cpt = -(-e // (NW * CHUNK))
    epad = NW * cpt * CHUNK

    src = jnp.concatenate(
        [edge_index[0], jnp.full((epad - e,), n, jnp.int32)]).reshape(
            NW, cpt, CHUNK)
    dst = jnp.concatenate(
        [edge_index[1], jnp.zeros((epad - e,), jnp.int32)]).reshape(
            NW, cpt, CHUNK)

    xp = jnp.pad(x, ((0, npad - n), (0, 0)))
    scatter = _scatter_partials(npad, cpt)

    z = _tc_project(xp, _pad_w(W1a), npad)
    p = scatter(z, src, dst)
    z = _tc_combine(p, _pad_b(b1a), _pad_w2(W2a), _pad_b(b2a), _pad_w2(W1b),
                    npad)
    p = scatter(z, src, dst)
    z = _tc_combine(p, _pad_b(b1b), _pad_w2(W2b), _pad_b(b2b), _pad_w2(W1c),
                    npad)
    p = scatter(z, src, dst)
    out = _tc_final(p, _pad_b(b1c), _pad_w2(W2c), _pad_b(b2c), npad, h_real)
    return out[:n, :h_real]


# speedup vs baseline: 11.0304x; 11.0304x over previous
import functools

import jax
import jax.numpy as jnp
from jax import lax
from jax.experimental import pallas as pl
from jax.experimental.pallas import tpu as pltpu
from jax.experimental.pallas import tpu_sc as plsc

HPAD = 32
NC, NS = 2, 16
NW = NC * NS
CHUNK = 128
NBUF = 4


def _scatter_partials(npad, cpt):
    rows_pt = npad // NS

    mesh = plsc.VectorSubcoreMesh(core_axis_name="c", subcore_axis_name="s")
    scratch = [
        pltpu.VMEM_SHARED((npad, HPAD), jnp.float32),
        pltpu.VMEM((cpt, CHUNK), jnp.int32),
        pltpu.VMEM((cpt, CHUNK), jnp.int32),
        pltpu.VMEM((NBUF, CHUNK, HPAD), jnp.float32),
    ] + [pltpu.SemaphoreType.DMA] * (2 * NBUF)

    @functools.partial(
        pl.kernel,
        out_type=jax.ShapeDtypeStruct((NC, npad, HPAD), jnp.float32),
        mesh=mesh,
        scratch_types=scratch,
        compiler_params=pltpu.CompilerParams(use_tc_tiling_on_sc=False),
    )
    def kern(z_hbm, src_hbm, dst_hbm, out_hbm, acc, src_v, dst_v, rows, *sems):
        gsem = sems[:NBUF]
        ssem = sems[NBUF:]
        c = lax.axis_index("c")
        s = lax.axis_index("s")
        wid = s * NC + c
        r0 = s * rows_pt

        pltpu.sync_copy(z_hbm.at[pl.ds(c * npad + r0, rows_pt)],
                        acc.at[pl.ds(r0, rows_pt)])
        pltpu.sync_copy(src_hbm.at[wid], src_v)
        pltpu.sync_copy(dst_hbm.at[wid], dst_v)

        for b in range(NBUF):
            pltpu.async_copy(z_hbm.at[src_v.at[b]], rows.at[b], gsem[b])

        plsc.subcore_barrier()

        def step(chunk, b, prefetch):
            pltpu.make_async_copy(
                z_hbm.at[src_v.at[chunk]], rows.at[b], gsem[b]).wait()
            pltpu.async_copy(
                rows.at[b], acc.at[dst_v.at[chunk]], ssem[b], add=True).wait()
            if prefetch:
                pltpu.async_copy(
                    z_hbm.at[src_v.at[chunk + NBUF]], rows.at[b], gsem[b])

        def body(jo, carry):
            for b in range(NBUF):
                step(jo * NBUF + b, b, True)
            return carry

        lax.fori_loop(0, cpt // NBUF - 1, body, 0)
        for b in range(NBUF):
            step(cpt - NBUF + b, b, False)

        plsc.subcore_barrier()
        pltpu.sync_copy(acc.at[pl.ds(r0, rows_pt)],
                        out_hbm.at[c, pl.ds(r0, rows_pt)])

    return kern


def _tc_project(x, w1, npad):
    def body(x_ref, w_ref, o_ref):
        z = jnp.dot(x_ref[...], w_ref[...], preferred_element_type=jnp.float32)
        o_ref[pl.ds(0, npad), :] = z
        o_ref[pl.ds(npad, npad), :] = jnp.zeros_like(z)

    return pl.pallas_call(
        body,
        out_shape=jax.ShapeDtypeStruct((2 * npad, HPAD), jnp.float32),
    )(x, w1)


def _tc_combine(p, b1, w2, b2, w1n, npad, n):
    def body(p_ref, b1_ref, w2_ref, b2_ref, w1n_ref, o_ref):
        u = jnp.maximum(p_ref[0] + p_ref[1] + b1_ref[...], 0.0)
        v = jnp.dot(u, w2_ref[...], preferred_element_type=jnp.float32) + b2_ref[...]
        h = jnp.maximum(v, 0.0)
        z = jnp.dot(h, w1n_ref[...], preferred_element_type=jnp.float32)
        row = lax.broadcasted_iota(jnp.int32, z.shape, 0)
        z = jnp.where(row < n, z, 0.0)
        o_ref[pl.ds(0, npad), :] = z
        o_ref[pl.ds(npad, npad), :] = jnp.zeros_like(z)

    return pl.pallas_call(
        body,
        out_shape=jax.ShapeDtypeStruct((2 * npad, HPAD), jnp.float32),
    )(p, b1, w2, b2, w1n)


def _tc_final(p, b1, w2, b2, npad, h_real):
    def body(p_ref, b1_ref, w2_ref, b2_ref, o_ref):
        u = jnp.maximum(p_ref[0] + p_ref[1] + b1_ref[...], 0.0)
        v = jnp.dot(u, w2_ref[...], preferred_element_type=jnp.float32) + b2_ref[...]
        h = jnp.maximum(v, 0.0)
        col = lax.broadcasted_iota(jnp.int32, h.shape, 1)
        hm = jnp.where(col < h_real, h, -jnp.inf)
        m = jnp.max(hm, axis=1, keepdims=True)
        lse = m + jnp.log(jnp.sum(jnp.exp(hm - m), axis=1, keepdims=True))
        o_ref[...] = h - lse

    return pl.pallas_call(
        body,
        out_shape=jax.ShapeDtypeStruct((npad, HPAD), jnp.float32),
    )(p, b1, w2, b2)


def _pad_w(w):
    return jnp.pad(w, ((0, 0), (0, HPAD - w.shape[1])))


def _pad_w2(w):
    return jnp.pad(w, ((0, HPAD - w.shape[0]), (0, HPAD - w.shape[1])))


def _pad_b(b):
    return jnp.pad(b, (0, HPAD - b.shape[0])).reshape(1, HPAD)


def kernel(x, edge_index, W1a, b1a, W2a, b2a, W1b, b1b, W2b, b2b,
           W1c, b1c, W2c, b2c):
    n, din = x.shape
    h_real = W1a.shape[1]
    e = edge_index.shape[1]

    npad = (n // 128 + 1) * 128
    cpt = NBUF * (-(-e // (NW * CHUNK * NBUF)))
    epad = NW * cpt * CHUNK

    src = jnp.concatenate(
        [edge_index[0], jnp.full((epad - e,), n, jnp.int32)]).reshape(
            NW, cpt, CHUNK)
    dst = jnp.concatenate(
        [edge_index[1], jnp.zeros((epad - e,), jnp.int32)]).reshape(
            NW, cpt, CHUNK)

    xp = jnp.pad(x, ((0, npad - n), (0, 0)))
    scatter = _scatter_partials(npad, cpt)

    z = _tc_project(xp, _pad_w(W1a), npad)
    p = scatter(z, src, dst)
    z = _tc_combine(p, _pad_b(b1a), _pad_w2(W2a), _pad_b(b2a), _pad_w2(W1b),
                    npad, n)
    p = scatter(z, src, dst)
    z = _tc_combine(p, _pad_b(b1b), _pad_w2(W2b), _pad_b(b2b), _pad_w2(W1c),
                    npad, n)
    p = scatter(z, src, dst)
    out = _tc_final(p, _pad_b(b1c), _pad_w2(W2c), _pad_b(b2c), npad, h_real)
    return out[:n, :h_real]


# speedup vs baseline: 11.1575x; 1.0115x over previous
import functools

import jax
import jax.numpy as jnp
from jax import lax
from jax.experimental import pallas as pl
from jax.experimental.pallas import tpu as pltpu
from jax.experimental.pallas import tpu_sc as plsc

HPAD = 32
NC, NS = 2, 16
NW = NC * NS
CHUNK = 256
NBUF = 4


def _scatter_partials(npad, cpt):
    rows_pt = npad // NS

    mesh = plsc.VectorSubcoreMesh(core_axis_name="c", subcore_axis_name="s")
    scratch = [
        pltpu.VMEM_SHARED((npad, HPAD), jnp.float32),
        pltpu.VMEM((cpt, CHUNK), jnp.int32),
        pltpu.VMEM((cpt, CHUNK), jnp.int32),
        pltpu.VMEM((NBUF, CHUNK, HPAD), jnp.float32),
    ] + [pltpu.SemaphoreType.DMA] * (2 * NBUF)

    @functools.partial(
        pl.kernel,
        out_type=jax.ShapeDtypeStruct((NC, npad, HPAD), jnp.float32),
        mesh=mesh,
        scratch_types=scratch,
        compiler_params=pltpu.CompilerParams(use_tc_tiling_on_sc=False),
    )
    def kern(z_hbm, src_hbm, dst_hbm, out_hbm, acc, src_v, dst_v, rows, *sems):
        gsem = sems[:NBUF]
        ssem = sems[NBUF:]
        c = lax.axis_index("c")
        s = lax.axis_index("s")
        wid = s * NC + c
        r0 = s * rows_pt

        pltpu.sync_copy(z_hbm.at[pl.ds(c * npad + r0, rows_pt)],
                        acc.at[pl.ds(r0, rows_pt)])
        pltpu.sync_copy(src_hbm.at[wid], src_v)
        pltpu.sync_copy(dst_hbm.at[wid], dst_v)

        for b in range(NBUF):
            pltpu.async_copy(z_hbm.at[src_v.at[b]], rows.at[b], gsem[b])

        plsc.subcore_barrier()

        def step(chunk, b, prefetch):
            pltpu.make_async_copy(
                z_hbm.at[src_v.at[chunk]], rows.at[b], gsem[b]).wait()
            pltpu.async_copy(
                rows.at[b], acc.at[dst_v.at[chunk]], ssem[b], add=True).wait()
            if prefetch:
                pltpu.async_copy(
                    z_hbm.at[src_v.at[chunk + NBUF]], rows.at[b], gsem[b])

        def body(jo, carry):
            for b in range(NBUF):
                step(jo * NBUF + b, b, True)
            return carry

        lax.fori_loop(0, cpt // NBUF - 1, body, 0)
        for b in range(NBUF):
            step(cpt - NBUF + b, b, False)

        plsc.subcore_barrier()
        pltpu.sync_copy(acc.at[pl.ds(r0, rows_pt)],
                        out_hbm.at[c, pl.ds(r0, rows_pt)])

    return kern


def _tc_project(x, w1, npad):
    def body(x_ref, w_ref, o_ref):
        z = jnp.dot(x_ref[...], w_ref[...], preferred_element_type=jnp.float32)
        o_ref[pl.ds(0, npad), :] = z
        o_ref[pl.ds(npad, npad), :] = jnp.zeros_like(z)

    return pl.pallas_call(
        body,
        out_shape=jax.ShapeDtypeStruct((2 * npad, HPAD), jnp.float32),
    )(x, w1)


def _tc_combine(p, b1, w2, b2, w1n, npad, n):
    def body(p_ref, b1_ref, w2_ref, b2_ref, w1n_ref, o_ref):
        u = jnp.maximum(p_ref[0] + p_ref[1] + b1_ref[...], 0.0)
        v = jnp.dot(u, w2_ref[...], preferred_element_type=jnp.float32) + b2_ref[...]
        h = jnp.maximum(v, 0.0)
        z = jnp.dot(h, w1n_ref[...], preferred_element_type=jnp.float32)
        row = lax.broadcasted_iota(jnp.int32, z.shape, 0)
        z = jnp.where(row < n, z, 0.0)
        o_ref[pl.ds(0, npad), :] = z
        o_ref[pl.ds(npad, npad), :] = jnp.zeros_like(z)

    return pl.pallas_call(
        body,
        out_shape=jax.ShapeDtypeStruct((2 * npad, HPAD), jnp.float32),
    )(p, b1, w2, b2, w1n)


def _tc_final(p, b1, w2, b2, npad, h_real):
    def body(p_ref, b1_ref, w2_ref, b2_ref, o_ref):
        u = jnp.maximum(p_ref[0] + p_ref[1] + b1_ref[...], 0.0)
        v = jnp.dot(u, w2_ref[...], preferred_element_type=jnp.float32) + b2_ref[...]
        h = jnp.maximum(v, 0.0)
        col = lax.broadcasted_iota(jnp.int32, h.shape, 1)
        hm = jnp.where(col < h_real, h, -jnp.inf)
        m = jnp.max(hm, axis=1, keepdims=True)
        lse = m + jnp.log(jnp.sum(jnp.exp(hm - m), axis=1, keepdims=True))
        o_ref[...] = h - lse

    return pl.pallas_call(
        body,
        out_shape=jax.ShapeDtypeStruct((npad, HPAD), jnp.float32),
    )(p, b1, w2, b2)


def _pad_w(w):
    return jnp.pad(w, ((0, 0), (0, HPAD - w.shape[1])))


def _pad_w2(w):
    return jnp.pad(w, ((0, HPAD - w.shape[0]), (0, HPAD - w.shape[1])))


def _pad_b(b):
    return jnp.pad(b, (0, HPAD - b.shape[0])).reshape(1, HPAD)


def kernel(x, edge_index, W1a, b1a, W2a, b2a, W1b, b1b, W2b, b2b,
           W1c, b1c, W2c, b2c):
    n, din = x.shape
    h_real = W1a.shape[1]
    e = edge_index.shape[1]

    npad = (n // 128 + 1) * 128
    cpt = NBUF * (-(-e // (NW * CHUNK * NBUF)))
    epad = NW * cpt * CHUNK

    src = jnp.concatenate(
        [edge_index[0], jnp.full((epad - e,), n, jnp.int32)]).reshape(
            NW, cpt, CHUNK)
    dst = jnp.concatenate(
        [edge_index[1], jnp.zeros((epad - e,), jnp.int32)]).reshape(
            NW, cpt, CHUNK)

    xp = jnp.pad(x, ((0, npad - n), (0, 0)))
    scatter = _scatter_partials(npad, cpt)

    z = _tc_project(xp, _pad_w(W1a), npad)
    p = scatter(z, src, dst)
    z = _tc_combine(p, _pad_b(b1a), _pad_w2(W2a), _pad_b(b2a), _pad_w2(W1b),
                    npad, n)
    p = scatter(z, src, dst)
    z = _tc_combine(p, _pad_b(b1b), _pad_w2(W2b), _pad_b(b2b), _pad_w2(W1c),
                    npad, n)
    p = scatter(z, src, dst)
    out = _tc_final(p, _pad_b(b1c), _pad_w2(W2c), _pad_b(b2c), npad, h_real)
    return out[:n, :h_real]


# speedup vs baseline: 11.1932x; 1.0032x over previous
import functools

import jax
import jax.numpy as jnp
from jax import lax
from jax.experimental import pallas as pl
from jax.experimental.pallas import tpu as pltpu
from jax.experimental.pallas import tpu_sc as plsc

HPAD = 32
NC, NS = 2, 16
NW = NC * NS
CHUNK = 256
D = 4
B = 8


def _scatter_partials(npad, cpt):
    rows_pt = npad // NS

    mesh = plsc.VectorSubcoreMesh(core_axis_name="c", subcore_axis_name="s")
    scratch = [
        pltpu.VMEM_SHARED((npad, HPAD), jnp.float32),
        pltpu.VMEM((cpt, CHUNK), jnp.int32),
        pltpu.VMEM((cpt, CHUNK), jnp.int32),
        pltpu.VMEM((B, CHUNK, HPAD), jnp.float32),
    ] + [pltpu.SemaphoreType.DMA] * (2 * B)

    @functools.partial(
        pl.kernel,
        out_type=jax.ShapeDtypeStruct((NC, npad, HPAD), jnp.float32),
        mesh=mesh,
        scratch_types=scratch,
        compiler_params=pltpu.CompilerParams(use_tc_tiling_on_sc=False),
    )
    def kern(z_hbm, src_hbm, dst_hbm, out_hbm, acc, src_v, dst_v, rows, *sems):
        gsem = sems[:B]
        ssem = sems[B:]
        c = lax.axis_index("c")
        s = lax.axis_index("s")
        wid = s * NC + c
        r0 = s * rows_pt

        pltpu.sync_copy(z_hbm.at[pl.ds(c * npad + r0, rows_pt)],
                        acc.at[pl.ds(r0, rows_pt)])
        pltpu.sync_copy(src_hbm.at[wid], src_v)
        pltpu.sync_copy(dst_hbm.at[wid], dst_v)

        for j in range(D):
            pltpu.async_copy(z_hbm.at[src_v.at[j]], rows.at[j], gsem[j])

        plsc.subcore_barrier()

        def step(chunk, b, wait_scatter, prefetch):
            pltpu.make_async_copy(
                z_hbm.at[src_v.at[chunk]], rows.at[b], gsem[b]).wait()
            pltpu.async_copy(
                rows.at[b], acc.at[dst_v.at[chunk]], ssem[b], add=True)
            if prefetch:
                j = chunk + D
                bj = (b + D) % B
                if wait_scatter:
                    pltpu.make_async_copy(
                        rows.at[bj], acc.at[dst_v.at[j - B]], ssem[bj]).wait()
                pltpu.async_copy(z_hbm.at[src_v.at[j]], rows.at[bj], gsem[bj])

        for k in range(D):
            step(k, k, False, True)

        def body(io, carry):
            for k in range(B):
                step(D + io * B + k, (D + k) % B, True, True)
            return carry

        lax.fori_loop(0, (cpt - 2 * D) // B, body, 0)
        for k in range(D):
            step(cpt - D + k, (cpt - D + k) % B, False, False)
        for k in range(B):
            pltpu.make_async_copy(
                rows.at[k], acc.at[dst_v.at[cpt - B + k]], ssem[k]).wait()

        plsc.subcore_barrier()
        pltpu.sync_copy(acc.at[pl.ds(r0, rows_pt)],
                        out_hbm.at[c, pl.ds(r0, rows_pt)])

    return kern


def _tc_project(x, w1, npad):
    def body(x_ref, w_ref, o_ref):
        z = jnp.dot(x_ref[...], w_ref[...], preferred_element_type=jnp.float32)
        o_ref[pl.ds(0, npad), :] = z
        o_ref[pl.ds(npad, npad), :] = jnp.zeros_like(z)

    return pl.pallas_call(
        body,
        out_shape=jax.ShapeDtypeStruct((2 * npad, HPAD), jnp.float32),
    )(x, w1)


def _tc_combine(p, b1, w2, b2, w1n, npad, n):
    def body(p_ref, b1_ref, w2_ref, b2_ref, w1n_ref, o_ref):
        u = jnp.maximum(p_ref[0] + p_ref[1] + b1_ref[...], 0.0)
        v = jnp.dot(u, w2_ref[...], preferred_element_type=jnp.float32) + b2_ref[...]
        h = jnp.maximum(v, 0.0)
        z = jnp.dot(h, w1n_ref[...], preferred_element_type=jnp.float32)
        row = lax.broadcasted_iota(jnp.int32, z.shape, 0)
        z = jnp.where(row < n, z, 0.0)
        o_ref[pl.ds(0, npad), :] = z
        o_ref[pl.ds(npad, npad), :] = jnp.zeros_like(z)

    return pl.pallas_call(
        body,
        out_shape=jax.ShapeDtypeStruct((2 * npad, HPAD), jnp.float32),
    )(p, b1, w2, b2, w1n)


def _tc_final(p, b1, w2, b2, npad, h_real):
    def body(p_ref, b1_ref, w2_ref, b2_ref, o_ref):
        u = jnp.maximum(p_ref[0] + p_ref[1] + b1_ref[...], 0.0)
        v = jnp.dot(u, w2_ref[...], preferred_element_type=jnp.float32) + b2_ref[...]
        h = jnp.maximum(v, 0.0)
        col = lax.broadcasted_iota(jnp.int32, h.shape, 1)
        hm = jnp.where(col < h_real, h, -jnp.inf)
        m = jnp.max(hm, axis=1, keepdims=True)
        lse = m + jnp.log(jnp.sum(jnp.exp(hm - m), axis=1, keepdims=True))
        o_ref[...] = h - lse

    return pl.pallas_call(
        body,
        out_shape=jax.ShapeDtypeStruct((npad, HPAD), jnp.float32),
    )(p, b1, w2, b2)


def _pad_w(w):
    return jnp.pad(w, ((0, 0), (0, HPAD - w.shape[1])))


def _pad_w2(w):
    return jnp.pad(w, ((0, HPAD - w.shape[0]), (0, HPAD - w.shape[1])))


def _pad_b(b):
    return jnp.pad(b, (0, HPAD - b.shape[0])).reshape(1, HPAD)


def kernel(x, edge_index, W1a, b1a, W2a, b2a, W1b, b1b, W2b, b2b,
           W1c, b1c, W2c, b2c):
    n, din = x.shape
    h_real = W1a.shape[1]
    e = edge_index.shape[1]

    npad = (n // 128 + 1) * 128
    cpt = B * (-(-e // (NW * CHUNK * B)))
    epad = NW * cpt * CHUNK

    src = jnp.concatenate(
        [edge_index[0], jnp.full((epad - e,), n, jnp.int32)]).reshape(
            NW, cpt, CHUNK)
    dst = jnp.concatenate(
        [edge_index[1], jnp.zeros((epad - e,), jnp.int32)]).reshape(
            NW, cpt, CHUNK)

    xp = jnp.pad(x, ((0, npad - n), (0, 0)))
    scatter = _scatter_partials(npad, cpt)

    z = _tc_project(xp, _pad_w(W1a), npad)
    p = scatter(z, src, dst)
    z = _tc_combine(p, _pad_b(b1a), _pad_w2(W2a), _pad_b(b2a), _pad_w2(W1b),
                    npad, n)
    p = scatter(z, src, dst)
    z = _tc_combine(p, _pad_b(b1b), _pad_w2(W2b), _pad_b(b2b), _pad_w2(W1c),
                    npad, n)
    p = scatter(z, src, dst)
    out = _tc_final(p, _pad_b(b1c), _pad_w2(W2c), _pad_b(b2c), npad, h_real)
    return out[:n, :h_real]


# speedup vs baseline: 18.9767x; 1.6954x over previous
import functools

import jax
import jax.numpy as jnp
from jax import lax
from jax.experimental import pallas as pl
from jax.experimental.pallas import tpu as pltpu
from jax.experimental.pallas import tpu_sc as plsc

HPAD = 32
NC, NS = 2, 16
NW = NC * NS
CHUNK = 256
D = 4
B = 8


def _scatter_partials(npad, cpt):
    rows_pt = npad // NS

    mesh = plsc.VectorSubcoreMesh(core_axis_name="c", subcore_axis_name="s")
    scratch = [
        pltpu.VMEM_SHARED((npad, HPAD), jnp.float32),
        pltpu.VMEM_SHARED((npad, HPAD), jnp.float32),
        pltpu.VMEM((cpt, CHUNK), jnp.int32),
        pltpu.VMEM((cpt, CHUNK), jnp.int32),
        pltpu.VMEM((B, CHUNK, HPAD), jnp.float32),
    ] + [pltpu.SemaphoreType.DMA] * (2 * B)

    @functools.partial(
        pl.kernel,
        out_type=jax.ShapeDtypeStruct((NC, npad, HPAD), jnp.float32),
        mesh=mesh,
        scratch_types=scratch,
        compiler_params=pltpu.CompilerParams(use_tc_tiling_on_sc=False),
    )
    def kern(z_hbm, src_hbm, dst_hbm, out_hbm, acc, z_sp, src_v, dst_v, rows,
             *sems):
        gsem = sems[:B]
        ssem = sems[B:]
        c = lax.axis_index("c")
        s = lax.axis_index("s")
        wid = s * NC + c
        r0 = s * rows_pt

        pltpu.sync_copy(z_hbm.at[pl.ds(c * npad + r0, rows_pt)],
                        acc.at[pl.ds(r0, rows_pt)])
        pltpu.sync_copy(z_hbm.at[pl.ds(r0, rows_pt)],
                        z_sp.at[pl.ds(r0, rows_pt)])
        pltpu.sync_copy(src_hbm.at[wid], src_v)
        pltpu.sync_copy(dst_hbm.at[wid], dst_v)

        plsc.subcore_barrier()

        for j in range(D):
            pltpu.async_copy(z_sp.at[src_v.at[j]], rows.at[j], gsem[j])

        def step(chunk, b, wait_scatter, prefetch):
            pltpu.make_async_copy(
                z_sp.at[src_v.at[chunk]], rows.at[b], gsem[b]).wait()
            pltpu.async_copy(
                rows.at[b], acc.at[dst_v.at[chunk]], ssem[b], add=True)
            if prefetch:
                j = chunk + D
                bj = (b + D) % B
                if wait_scatter:
                    pltpu.make_async_copy(
                        rows.at[bj], acc.at[dst_v.at[j - B]], ssem[bj]).wait()
                pltpu.async_copy(z_sp.at[src_v.at[j]], rows.at[bj], gsem[bj])

        for k in range(D):
            step(k, k, False, True)

        def body(io, carry):
            for k in range(B):
                step(D + io * B + k, (D + k) % B, True, True)
            return carry

        lax.fori_loop(0, (cpt - 2 * D) // B, body, 0)
        for k in range(D):
            step(cpt - D + k, (cpt - D + k) % B, False, False)
        for k in range(B):
            pltpu.make_async_copy(
                rows.at[k], acc.at[dst_v.at[cpt - B + k]], ssem[k]).wait()

        plsc.subcore_barrier()
        pltpu.sync_copy(acc.at[pl.ds(r0, rows_pt)],
                        out_hbm.at[c, pl.ds(r0, rows_pt)])

    return kern


def _tc_project(x, w1, npad):
    def body(x_ref, w_ref, o_ref):
        z = jnp.dot(x_ref[...], w_ref[...], preferred_element_type=jnp.float32)
        o_ref[pl.ds(0, npad), :] = z
        o_ref[pl.ds(npad, npad), :] = jnp.zeros_like(z)

    return pl.pallas_call(
        body,
        out_shape=jax.ShapeDtypeStruct((2 * npad, HPAD), jnp.float32),
    )(x, w1)


def _tc_combine(p, b1, w2, b2, w1n, npad, n):
    def body(p_ref, b1_ref, w2_ref, b2_ref, w1n_ref, o_ref):
        u = jnp.maximum(p_ref[0] + p_ref[1] + b1_ref[...], 0.0)
        v = jnp.dot(u, w2_ref[...], preferred_element_type=jnp.float32) + b2_ref[...]
        h = jnp.maximum(v, 0.0)
        z = jnp.dot(h, w1n_ref[...], preferred_element_type=jnp.float32)
        row = lax.broadcasted_iota(jnp.int32, z.shape, 0)
        z = jnp.where(row < n, z, 0.0)
        o_ref[pl.ds(0, npad), :] = z
        o_ref[pl.ds(npad, npad), :] = jnp.zeros_like(z)

    return pl.pallas_call(
        body,
        out_shape=jax.ShapeDtypeStruct((2 * npad, HPAD), jnp.float32),
    )(p, b1, w2, b2, w1n)


def _tc_final(p, b1, w2, b2, npad, h_real):
    def body(p_ref, b1_ref, w2_ref, b2_ref, o_ref):
        u = jnp.maximum(p_ref[0] + p_ref[1] + b1_ref[...], 0.0)
        v = jnp.dot(u, w2_ref[...], preferred_element_type=jnp.float32) + b2_ref[...]
        h = jnp.maximum(v, 0.0)
        col = lax.broadcasted_iota(jnp.int32, h.shape, 1)
        hm = jnp.where(col < h_real, h, -jnp.inf)
        m = jnp.max(hm, axis=1, keepdims=True)
        lse = m + jnp.log(jnp.sum(jnp.exp(hm - m), axis=1, keepdims=True))
        o_ref[...] = h - lse

    return pl.pallas_call(
        body,
        out_shape=jax.ShapeDtypeStruct((npad, HPAD), jnp.float32),
    )(p, b1, w2, b2)


def _pad_w(w):
    return jnp.pad(w, ((0, 0), (0, HPAD - w.shape[1])))


def _pad_w2(w):
    return jnp.pad(w, ((0, HPAD - w.shape[0]), (0, HPAD - w.shape[1])))


def _pad_b(b):
    return jnp.pad(b, (0, HPAD - b.shape[0])).reshape(1, HPAD)


def kernel(x, edge_index, W1a, b1a, W2a, b2a, W1b, b1b, W2b, b2b,
           W1c, b1c, W2c, b2c):
    n, din = x.shape
    h_real = W1a.shape[1]
    e = edge_index.shape[1]

    npad = (n // 128 + 1) * 128
    cpt = B * (-(-e // (NW * CHUNK * B)))
    epad = NW * cpt * CHUNK

    src = jnp.concatenate(
        [edge_index[0], jnp.full((epad - e,), n, jnp.int32)]).reshape(
            NW, cpt, CHUNK)
    dst = jnp.concatenate(
        [edge_index[1], jnp.zeros((epad - e,), jnp.int32)]).reshape(
            NW, cpt, CHUNK)

    xp = jnp.pad(x, ((0, npad - n), (0, 0)))
    scatter = _scatter_partials(npad, cpt)

    z = _tc_project(xp, _pad_w(W1a), npad)
    p = scatter(z, src, dst)
    z = _tc_combine(p, _pad_b(b1a), _pad_w2(W2a), _pad_b(b2a), _pad_w2(W1b),
                    npad, n)
    p = scatter(z, src, dst)
    z = _tc_combine(p, _pad_b(b1b), _pad_w2(W2b), _pad_b(b2b), _pad_w2(W1c),
                    npad, n)
    p = scatter(z, src, dst)
    out = _tc_final(p, _pad_b(b1c), _pad_w2(W2c), _pad_b(b2c), npad, h_real)
    return out[:n, :h_real]
